# baseline (speedup 1.0000x reference)
import functools

import jax
import jax.numpy as jnp
from jax import lax
from jax.experimental import pallas as pl
from jax.experimental.pallas import tpu as pltpu
from jax.experimental.pallas import tpu_sc as plsc

_B = 4096
_T = 50
_D = 64
_N = _B * _T
_NW = 32
_PER_W = _N // _NW
_CHUNK = 128
_NCH = _PER_W // _CHUNK

_mesh = plsc.VectorSubcoreMesh(core_axis_name="c", subcore_axis_name="s")


@functools.partial(
    pl.kernel,
    mesh=_mesh,
    out_type=jax.ShapeDtypeStruct((_N, _D), jnp.float32),
    scratch_types=[
        pltpu.VMEM((_PER_W,), jnp.int32),
        pltpu.VMEM((_CHUNK, _D), jnp.float32),
        pltpu.SemaphoreType.DMA,
    ],
    compiler_params=pltpu.CompilerParams(use_tc_tiling_on_sc=False),
)
def _gather_kernel(idx_hbm, table_hbm, out_hbm, idx_v, rows_v, sem):
    wid = lax.axis_index("s") * 2 + lax.axis_index("c")
    base = wid * _PER_W
    pltpu.sync_copy(idx_hbm.at[pl.ds(base, _PER_W)], idx_v)

    def body(j, carry):
        off = j * _CHUNK
        pltpu.async_copy(
            table_hbm.at[idx_v.at[pl.ds(off, _CHUNK)]], rows_v, sem
        ).wait()
        pltpu.sync_copy(rows_v, out_hbm.at[pl.ds(base + off, _CHUNK)])
        return carry

    lax.fori_loop(0, _NCH, body, 0)


def kernel(captions, lengths, emb_weight):
    idx = captions.reshape(_N)
    out = _gather_kernel(idx, emb_weight)
    return out.reshape(_B, _T, _D), lengths


# speedup vs baseline: 1.0364x; 1.0364x over previous
import functools

import jax
import jax.numpy as jnp
from jax import lax
from jax.experimental import pallas as pl
from jax.experimental.pallas import tpu as pltpu
from jax.experimental.pallas import tpu_sc as plsc

_B = 4096
_T = 50
_D = 64
_N = _B * _T
_NW = 32
_PER_W = _N // _NW
_CHUNK = 128
_NCH = _PER_W // _CHUNK

_mesh = plsc.VectorSubcoreMesh(core_axis_name="c", subcore_axis_name="s")


@functools.partial(
    pl.kernel,
    mesh=_mesh,
    out_type=jax.ShapeDtypeStruct((_N, _D), jnp.float32),
    scratch_types=[
        pltpu.VMEM((_PER_W,), jnp.int32),
        [pltpu.VMEM((_CHUNK, _D), jnp.float32) for _ in range(2)],
        pltpu.SemaphoreType.DMA((2,)),
        pltpu.SemaphoreType.DMA((2,)),
    ],
    compiler_params=pltpu.CompilerParams(use_tc_tiling_on_sc=False),
)
def _gather_kernel(idx_hbm, table_hbm, out_hbm, idx_v, rows_v, sem, osem):
    wid = lax.axis_index("s") * 2 + lax.axis_index("c")
    base = wid * _PER_W
    pltpu.sync_copy(idx_hbm.at[pl.ds(base, _PER_W)], idx_v)

    def gather(j):
        return pltpu.make_async_copy(
            table_hbm.at[idx_v.at[pl.ds(j * _CHUNK, _CHUNK)]], rows_v[j % 2],
            sem.at[j % 2],
        )

    def put(j):
        return pltpu.make_async_copy(
            rows_v[j % 2], out_hbm.at[pl.ds(base + j * _CHUNK, _CHUNK)],
            osem.at[j % 2],
        )

    gather(0).start()
    gather(1).start()
    for j in range(_NCH):
        gather(j).wait()
        put(j).start()
        jn = j + 2
        if jn < _NCH:
            put(j).wait()
            gather(jn).start()
        else:
            put(j).wait()


def kernel(captions, lengths, emb_weight):
    idx = captions.reshape(_N)
    out = _gather_kernel(idx, emb_weight)
    return out.reshape(_B, _T, _D), lengths
